# Initial kernel scaffold; baseline (speedup 1.0000x reference)
import jax
import jax.numpy as jnp
from jax.experimental import pallas as pl


def kernel(class_seq, api_seq, class_table, api_table):
    raise NotImplementedError("write your pallas kernel here")


# speedup vs baseline: 3.3470x; 3.3470x over previous
import functools
import math

import jax
import jax.numpy as jnp
from jax import lax
from jax.experimental import pallas as pl
from jax.experimental.pallas import tpu as pltpu
from jax.experimental.pallas import tpu_sc as plsc

NUM_CORES = 2
NUM_SUBCORES = 16
NUM_WORKERS = NUM_CORES * NUM_SUBCORES
CHUNK = 128


def _scale_body(scale, zero_starts, tbl_ref, out_ref):
    x = tbl_ref[...]
    rows = lax.broadcasted_iota(jnp.int32, x.shape, 0)
    cols = lax.broadcasted_iota(jnp.int32, x.shape, 1)
    mask = jnp.zeros(x.shape, jnp.bool_)
    for r0, ncols in zero_starts:
        mask = mask | ((rows == r0) & (cols < ncols))
    out_ref[...] = jnp.where(mask, 0.0, x * scale)


def _scale_table(flat_tbl, scale, zero_starts):
    return pl.pallas_call(
        functools.partial(_scale_body, scale, zero_starts),
        out_shape=jax.ShapeDtypeStruct(flat_tbl.shape, flat_tbl.dtype),
    )(flat_tbl)


def _gather_body(rows_per_w, tbl_hbm, idx_hbm, out_hbm, idx_v, rows_v, sem):
    wid = lax.axis_index("s") * NUM_CORES + lax.axis_index("c")
    n_chunks = rows_per_w // CHUNK

    def body(c, carry):
        base = wid * rows_per_w + c * CHUNK
        pltpu.sync_copy(idx_hbm.at[pl.ds(base, CHUNK)], idx_v)
        pltpu.async_copy(tbl_hbm.at[idx_v], rows_v, sem).wait()
        pltpu.sync_copy(rows_v, out_hbm.at[pl.ds(base, CHUNK)])
        return carry

    lax.fori_loop(0, n_chunks, body, 0)


def _gather_rows(tbl, idx2, n_rows, d):
    rows_per_w = n_rows // NUM_WORKERS
    mesh = plsc.VectorSubcoreMesh(
        core_axis_name="c", subcore_axis_name="s",
        num_cores=NUM_CORES, num_subcores=NUM_SUBCORES)
    return pl.kernel(
        functools.partial(_gather_body, rows_per_w),
        out_type=jax.ShapeDtypeStruct((n_rows, d), jnp.float32),
        mesh=mesh,
        scratch_types=[
            pltpu.VMEM((CHUNK,), jnp.int32),
            pltpu.VMEM((CHUNK, d), jnp.float32),
            pltpu.SemaphoreType.DMA,
        ],
        compiler_params=pltpu.CompilerParams(use_tc_tiling_on_sc=False),
    )(tbl, idx2)


def kernel(class_seq, api_seq, class_table, api_table):
    b, s = class_seq.shape
    av, da = api_table.shape
    cv, dc = class_table.shape
    d_model = dc + da
    scale = math.sqrt(float(d_model))
    t = b * s

    big = jnp.concatenate([api_table, class_table], axis=0)
    n_el = (av + cv) * da
    flat = big.reshape(n_el // 128, 128)
    zero_starts = [(0, da), (av * da // 128, da)]
    flat_scaled = _scale_table(flat, scale, zero_starts)
    tbl = flat_scaled.reshape(av + cv, da)

    idx2 = jnp.stack(
        [class_seq.reshape(t).astype(jnp.int32) + av,
         api_seq.reshape(t).astype(jnp.int32)], axis=1).reshape(2 * t)

    out2 = _gather_rows(tbl, idx2, 2 * t, da)
    return out2.reshape(b, s, d_model)


# speedup vs baseline: 4.2310x; 1.2641x over previous
import functools
import math

import jax
import jax.numpy as jnp
from jax import lax
from jax.experimental import pallas as pl
from jax.experimental.pallas import tpu as pltpu
from jax.experimental.pallas import tpu_sc as plsc

NUM_CORES = 2
NUM_SUBCORES = 16
NUM_WORKERS = NUM_CORES * NUM_SUBCORES
CHUNK = 128


def _scale_body(scale, zero_starts, tbl_ref, out_ref):
    x = tbl_ref[...]
    rows = lax.broadcasted_iota(jnp.int32, x.shape, 0)
    cols = lax.broadcasted_iota(jnp.int32, x.shape, 1)
    mask = jnp.zeros(x.shape, jnp.bool_)
    for r0, ncols in zero_starts:
        mask = mask | ((rows == r0) & (cols < ncols))
    out_ref[...] = jnp.where(mask, 0.0, x * scale)


def _scale_table(flat_tbl, scale, zero_starts):
    return pl.pallas_call(
        functools.partial(_scale_body, scale, zero_starts),
        out_shape=jax.ShapeDtypeStruct(flat_tbl.shape, flat_tbl.dtype),
    )(flat_tbl)


G = 640
NCH = G // CHUNK


def _gather_body(rows_per_w, tbl_hbm, idx_hbm, out_hbm, idx_v, rows_v,
                 si0, si1, sg, so0, so1):
    wid = lax.axis_index("s") * NUM_CORES + lax.axis_index("c")
    w0 = wid * rows_per_w
    ngroups = rows_per_w // G
    si = (si0, si1)
    so = (so0, so1)

    def idx_copy(g, s):
        pltpu.async_copy(idx_hbm.at[pl.ds(w0 + g * G, G)], idx_v.at[s], si[s])

    def process(g, s, wait_out, prefetch):
        if wait_out:
            pltpu.make_async_copy(
                rows_v.at[s], out_hbm.at[pl.ds(w0 + g * G, G)], so[s]).wait()
        pltpu.make_async_copy(
            idx_hbm.at[pl.ds(w0 + g * G, G)], idx_v.at[s], si[s]).wait()
        descs = [
            pltpu.async_copy(
                tbl_hbm.at[idx_v.at[s, pl.ds(j * CHUNK, CHUNK)]],
                rows_v.at[s, pl.ds(j * CHUNK, CHUNK), :], sg)
            for j in range(NCH)
        ]
        for dsc in descs:
            dsc.wait()
        pltpu.async_copy(rows_v.at[s], out_hbm.at[pl.ds(w0 + g * G, G)], so[s])
        if prefetch:
            idx_copy(g + 2, s)

    for s in range(2):
        idx_copy(s, s)
    for g in range(2):
        process(g, g, wait_out=False, prefetch=True)

    def outer(g2, carry):
        for s in range(2):
            process(2 * g2 + s, s, wait_out=True, prefetch=True)
        return carry

    lax.fori_loop(1, ngroups // 2 - 1, outer, 0)

    for g in (ngroups - 2, ngroups - 1):
        process(g, g % 2, wait_out=True, prefetch=False)
    for g in (ngroups - 2, ngroups - 1):
        pltpu.make_async_copy(
            rows_v.at[g % 2], out_hbm.at[pl.ds(w0 + g * G, G)], so[g % 2]).wait()


def _gather_rows(tbl, idx2, n_rows, d):
    rows_per_w = n_rows // NUM_WORKERS
    mesh = plsc.VectorSubcoreMesh(
        core_axis_name="c", subcore_axis_name="s",
        num_cores=NUM_CORES, num_subcores=NUM_SUBCORES)
    return pl.kernel(
        functools.partial(_gather_body, rows_per_w),
        out_type=jax.ShapeDtypeStruct((n_rows, d), jnp.float32),
        mesh=mesh,
        scratch_types=[
            pltpu.VMEM((2, G), jnp.int32),
            pltpu.VMEM((2, G, d), jnp.float32),
            pltpu.SemaphoreType.DMA,
            pltpu.SemaphoreType.DMA,
            pltpu.SemaphoreType.DMA,
            pltpu.SemaphoreType.DMA,
            pltpu.SemaphoreType.DMA,
        ],
        compiler_params=pltpu.CompilerParams(use_tc_tiling_on_sc=False),
    )(tbl, idx2)


def kernel(class_seq, api_seq, class_table, api_table):
    b, s = class_seq.shape
    av, da = api_table.shape
    cv, dc = class_table.shape
    d_model = dc + da
    scale = math.sqrt(float(d_model))
    t = b * s

    big = jnp.concatenate([api_table, class_table], axis=0)
    n_el = (av + cv) * da
    flat = big.reshape(n_el // 128, 128)
    zero_starts = [(0, da), (av * da // 128, da)]
    flat_scaled = _scale_table(flat, scale, zero_starts)
    tbl = flat_scaled.reshape(av + cv, da)

    idx2 = jnp.stack(
        [class_seq.reshape(t).astype(jnp.int32) + av,
         api_seq.reshape(t).astype(jnp.int32)], axis=1).reshape(2 * t)

    out2 = _gather_rows(tbl, idx2, 2 * t, da)
    return out2.reshape(b, s, d_model)


# speedup vs baseline: 13.5524x; 3.2031x over previous
import functools
import math

import jax
import jax.numpy as jnp
from jax import lax
from jax.experimental import pallas as pl
from jax.experimental.pallas import tpu as pltpu
from jax.experimental.pallas import tpu_sc as plsc

NUM_CORES = 2
NUM_SUBCORES = 16
NUM_WORKERS = NUM_CORES * NUM_SUBCORES
LANES = 16

CHUNK = 128
G = 640
NCH = G // CHUNK
NT = G // 2

SCALE_ROWS = 625


def _scale_body(scale, av, cv, api_hbm, cls_hbm, out_hbm, buf, zbuf, sd):
    wid = lax.axis_index("s") * NUM_CORES + lax.axis_index("c")

    def scale_buf(nrows):
        def body(i, carry):
            for j in range(64 // LANES):
                v = buf[i, pl.ds(j * LANES, LANES)]
                buf[i, pl.ds(j * LANES, LANES)] = v * scale
            return carry
        lax.fori_loop(0, nrows, body, 0)

    def do_span(src, src_base, dst_base, nrows):
        pltpu.async_copy(
            src.at[pl.ds(src_base, nrows)],
            buf.at[pl.ds(0, nrows), :], sd).wait()
        scale_buf(nrows)
        pltpu.async_copy(
            buf.at[pl.ds(0, nrows), :],
            out_hbm.at[pl.ds(dst_base, nrows)], sd).wait()

    def zero_row(dst_row):
        zero = jnp.zeros((LANES,), jnp.float32)
        for i in range(64 // LANES):
            zbuf[0, pl.ds(i * LANES, LANES)] = zero
        pltpu.async_copy(zbuf, out_hbm.at[pl.ds(dst_row, 1)], sd).wait()

    rows_w = av // NUM_WORKERS
    nch = rows_w // SCALE_ROWS
    base = wid * rows_w

    def api_chunk(c, carry):
        b = base + c * SCALE_ROWS
        do_span(api_hbm, b, b, SCALE_ROWS)
        return carry

    lax.fori_loop(0, nch, api_chunk, 0)
    rem = rows_w - nch * SCALE_ROWS
    if rem:
        do_span(api_hbm, base + nch * SCALE_ROWS, base + nch * SCALE_ROWS, rem)

    @pl.when(wid == 0)
    def _():
        zero_row(0)

    @pl.when(wid == NUM_WORKERS - 1)
    def _():
        nfull = cv // SCALE_ROWS
        for c in range(nfull):
            do_span(cls_hbm, c * SCALE_ROWS, av + c * SCALE_ROWS, SCALE_ROWS)
        crem = cv - nfull * SCALE_ROWS
        if crem:
            do_span(cls_hbm, nfull * SCALE_ROWS, av + nfull * SCALE_ROWS, crem)
        zero_row(av)


def _build_table(api_table, class_table, scale):
    av, d = api_table.shape
    cv = class_table.shape[0]
    mesh = plsc.VectorSubcoreMesh(
        core_axis_name="c", subcore_axis_name="s",
        num_cores=NUM_CORES, num_subcores=NUM_SUBCORES)
    return pl.kernel(
        functools.partial(_scale_body, scale, av, cv),
        out_type=jax.ShapeDtypeStruct((av + cv, d), jnp.float32),
        mesh=mesh,
        scratch_types=[
            pltpu.VMEM((SCALE_ROWS, d), jnp.float32),
            pltpu.VMEM((1, d), jnp.float32),
            pltpu.SemaphoreType.DMA,
        ],
        compiler_params=pltpu.CompilerParams(use_tc_tiling_on_sc=False, needs_layout_passes=False),
    )(api_table, class_table)


def _gather_body(rows_per_w, av, tbl_hbm, cls_hbm, api_hbm, out_hbm,
                 cls_v, api_v, idx_v, rows_v, si0, si1, sg, so0, so1):
    wid = lax.axis_index("s") * NUM_CORES + lax.axis_index("c")
    w0 = wid * rows_per_w
    t0 = wid * (rows_per_w // 2)
    ngroups = rows_per_w // G
    si = (si0, si1)
    so = (so0, so1)
    pos0 = lax.iota(jnp.int32, LANES) * 2

    def idx_copy(g, s):
        tb = t0 + g * NT
        pltpu.async_copy(cls_hbm.at[pl.ds(tb, NT)], cls_v.at[s], si[s])
        pltpu.async_copy(api_hbm.at[pl.ds(tb, NT)], api_v.at[s], si[s])

    def interleave(s):
        def body(i, carry):
            c = cls_v[s, pl.ds(i * LANES, LANES)] + av
            a = api_v[s, pl.ds(i * LANES, LANES)]
            pos = pos0 + i * (2 * LANES)
            plsc.store_scatter(idx_v.at[s], [pos], c)
            plsc.store_scatter(idx_v.at[s], [pos + 1], a)
            return carry
        lax.fori_loop(0, NT // LANES, body, 0)

    def process(g, s, wait_out, prefetch):
        if wait_out:
            pltpu.make_async_copy(
                rows_v.at[s], out_hbm.at[pl.ds(w0 + g * G, G)], so[s]).wait()
        tb = t0 + g * NT
        pltpu.make_async_copy(
            cls_hbm.at[pl.ds(tb, NT)], cls_v.at[s], si[s]).wait()
        pltpu.make_async_copy(
            api_hbm.at[pl.ds(tb, NT)], api_v.at[s], si[s]).wait()
        interleave(s)
        descs = [
            pltpu.async_copy(
                tbl_hbm.at[idx_v.at[s, pl.ds(j * CHUNK, CHUNK)]],
                rows_v.at[s, pl.ds(j * CHUNK, CHUNK), :], sg)
            for j in range(NCH)
        ]
        for dsc in descs:
            dsc.wait()
        pltpu.async_copy(rows_v.at[s], out_hbm.at[pl.ds(w0 + g * G, G)], so[s])
        if prefetch:
            idx_copy(g + 2, s)

    for s in range(2):
        idx_copy(s, s)
    for g in range(2):
        process(g, g, wait_out=False, prefetch=True)

    def outer(g2, carry):
        for s in range(2):
            process(2 * g2 + s, s, wait_out=True, prefetch=True)
        return carry

    lax.fori_loop(1, ngroups // 2 - 1, outer, 0)

    for g in (ngroups - 2, ngroups - 1):
        process(g, g % 2, wait_out=True, prefetch=False)
    for g in (ngroups - 2, ngroups - 1):
        pltpu.make_async_copy(
            rows_v.at[g % 2], out_hbm.at[pl.ds(w0 + g * G, G)], so[g % 2]).wait()


def _gather_rows(tbl, cls_flat, api_flat, av, n_rows, d):
    rows_per_w = n_rows // NUM_WORKERS
    mesh = plsc.VectorSubcoreMesh(
        core_axis_name="c", subcore_axis_name="s",
        num_cores=NUM_CORES, num_subcores=NUM_SUBCORES)
    return pl.kernel(
        functools.partial(_gather_body, rows_per_w, av),
        out_type=jax.ShapeDtypeStruct((n_rows, d), jnp.float32),
        mesh=mesh,
        scratch_types=[
            pltpu.VMEM((2, NT), jnp.int32),
            pltpu.VMEM((2, NT), jnp.int32),
            pltpu.VMEM((2, G), jnp.int32),
            pltpu.VMEM((2, G, d), jnp.float32),
            pltpu.SemaphoreType.DMA,
            pltpu.SemaphoreType.DMA,
            pltpu.SemaphoreType.DMA,
            pltpu.SemaphoreType.DMA,
            pltpu.SemaphoreType.DMA,
        ],
        compiler_params=pltpu.CompilerParams(use_tc_tiling_on_sc=False, needs_layout_passes=False),
    )(tbl, cls_flat, api_flat)


def kernel(class_seq, api_seq, class_table, api_table):
    b, s = class_seq.shape
    av, da = api_table.shape
    d_model = class_table.shape[1] + da
    scale = math.sqrt(float(d_model))
    t = b * s

    tbl = _build_table(api_table, class_table, scale)
    cls_flat = class_seq.reshape(t).astype(jnp.int32)
    api_flat = api_seq.reshape(t).astype(jnp.int32)
    out2 = _gather_rows(tbl, cls_flat, api_flat, av, 2 * t, da)
    return out2.reshape(b, s, d_model)


# speedup vs baseline: 13.7615x; 1.0154x over previous
import functools
import math

import jax
import jax.numpy as jnp
from jax import lax
from jax.experimental import pallas as pl
from jax.experimental.pallas import tpu as pltpu
from jax.experimental.pallas import tpu_sc as plsc

NUM_CORES = 2
NUM_SUBCORES = 16
NUM_WORKERS = NUM_CORES * NUM_SUBCORES
LANES = 16

CHUNK = 128
G = 640
NCH = G // CHUNK
NT = G // 2

SCALE_ROWS = 625


def _scale_body(scale, av, cv, api_hbm, cls_hbm, out_hbm, buf, zbuf, sd):
    wid = lax.axis_index("s") * NUM_CORES + lax.axis_index("c")

    def scale_buf(nrows):
        def body(i, carry):
            for j in range(64 // LANES):
                v = buf[i, pl.ds(j * LANES, LANES)]
                buf[i, pl.ds(j * LANES, LANES)] = v * scale
            return carry
        lax.fori_loop(0, nrows, body, 0)

    def do_span(src, src_base, dst_base, nrows):
        pltpu.async_copy(
            src.at[pl.ds(src_base, nrows)],
            buf.at[pl.ds(0, nrows), :], sd).wait()
        scale_buf(nrows)
        pltpu.async_copy(
            buf.at[pl.ds(0, nrows), :],
            out_hbm.at[pl.ds(dst_base, nrows)], sd).wait()

    def zero_row(dst_row):
        zero = jnp.zeros((LANES,), jnp.float32)
        for i in range(64 // LANES):
            zbuf[0, pl.ds(i * LANES, LANES)] = zero
        pltpu.async_copy(zbuf, out_hbm.at[pl.ds(dst_row, 1)], sd).wait()

    rows_w = av // NUM_WORKERS
    nch = rows_w // SCALE_ROWS
    base = wid * rows_w

    def api_chunk(c, carry):
        b = base + c * SCALE_ROWS
        do_span(api_hbm, b, b, SCALE_ROWS)
        return carry

    lax.fori_loop(0, nch, api_chunk, 0)
    rem = rows_w - nch * SCALE_ROWS
    if rem:
        do_span(api_hbm, base + nch * SCALE_ROWS, base + nch * SCALE_ROWS, rem)

    @pl.when(wid == 0)
    def _():
        zero_row(0)

    @pl.when(wid == NUM_WORKERS - 1)
    def _():
        nfull = cv // SCALE_ROWS
        for c in range(nfull):
            do_span(cls_hbm, c * SCALE_ROWS, av + c * SCALE_ROWS, SCALE_ROWS)
        crem = cv - nfull * SCALE_ROWS
        if crem:
            do_span(cls_hbm, nfull * SCALE_ROWS, av + nfull * SCALE_ROWS, crem)
        zero_row(av)


def _build_table(api_table, class_table, scale):
    av, d = api_table.shape
    cv = class_table.shape[0]
    mesh = plsc.VectorSubcoreMesh(
        core_axis_name="c", subcore_axis_name="s",
        num_cores=NUM_CORES, num_subcores=NUM_SUBCORES)
    return pl.kernel(
        functools.partial(_scale_body, scale, av, cv),
        out_type=jax.ShapeDtypeStruct((av + cv, d), jnp.float32),
        mesh=mesh,
        scratch_types=[
            pltpu.VMEM((SCALE_ROWS, d), jnp.float32),
            pltpu.VMEM((1, d), jnp.float32),
            pltpu.SemaphoreType.DMA,
        ],
        compiler_params=pltpu.CompilerParams(use_tc_tiling_on_sc=False, needs_layout_passes=False),
    )(api_table, class_table)


def _gather_body(rows_per_w, av, tbl_hbm, cls_hbm, api_hbm, out_hbm,
                 cls_v, api_v, idx_v, rows_v, si0, si1, sg0, sg1, so0, so1):
    wid = lax.axis_index("s") * NUM_CORES + lax.axis_index("c")
    w0 = wid * rows_per_w
    t0 = wid * (rows_per_w // 2)
    ngroups = rows_per_w // G
    si = (si0, si1)
    sg = (sg0, sg1)
    so = (so0, so1)
    pos0 = lax.iota(jnp.int32, LANES) * 2

    def idx_copy(g, s):
        tb = t0 + g * NT
        pltpu.async_copy(cls_hbm.at[pl.ds(tb, NT)], cls_v.at[s], si[s])
        pltpu.async_copy(api_hbm.at[pl.ds(tb, NT)], api_v.at[s], si[s])

    def interleave(s):
        def body(i, carry):
            c = cls_v[s, pl.ds(i * LANES, LANES)] + av
            a = api_v[s, pl.ds(i * LANES, LANES)]
            pos = pos0 + i * (2 * LANES)
            plsc.store_scatter(idx_v.at[s], [pos], c)
            plsc.store_scatter(idx_v.at[s], [pos + 1], a)
            return carry
        lax.fori_loop(0, NT // LANES, body, 0)

    def fire_group(g, s):
        tb = t0 + g * NT
        pltpu.make_async_copy(
            cls_hbm.at[pl.ds(tb, NT)], cls_v.at[s], si[s]).wait()
        pltpu.make_async_copy(
            api_hbm.at[pl.ds(tb, NT)], api_v.at[s], si[s]).wait()
        interleave(s)
        for j in range(NCH):
            pltpu.async_copy(
                tbl_hbm.at[idx_v.at[s, pl.ds(j * CHUNK, CHUNK)]],
                rows_v.at[s, pl.ds(j * CHUNK, CHUNK), :], sg[s])

    def drain_gathers(s):
        for j in range(NCH):
            pltpu.make_async_copy(
                tbl_hbm.at[idx_v.at[s, pl.ds(j * CHUNK, CHUNK)]],
                rows_v.at[s, pl.ds(j * CHUNK, CHUNK), :], sg[s]).wait()

    def fire_out(g, s):
        pltpu.async_copy(rows_v.at[s], out_hbm.at[pl.ds(w0 + g * G, G)], so[s])

    def wait_out(g, s):
        pltpu.make_async_copy(
            rows_v.at[s], out_hbm.at[pl.ds(w0 + g * G, G)], so[s]).wait()

    for s in range(2):
        idx_copy(s, s)
    fire_group(0, 0)
    idx_copy(2, 0)
    fire_group(1, 1)
    drain_gathers(0)
    fire_out(0, 0)
    idx_copy(3, 1)

    def outer(g2, carry):
        for s in range(2):
            g = 2 * g2 + s
            wait_out(g - 2, s)
            fire_group(g, s)
            drain_gathers(1 - s)
            fire_out(g - 1, 1 - s)
            idx_copy(g + 2, s)
        return carry

    lax.fori_loop(1, ngroups // 2 - 1, outer, 0)

    for g in (ngroups - 2, ngroups - 1):
        s = g % 2
        wait_out(g - 2, s)
        fire_group(g, s)
        drain_gathers(1 - s)
        fire_out(g - 1, 1 - s)
    drain_gathers((ngroups - 1) % 2)
    fire_out(ngroups - 1, (ngroups - 1) % 2)
    for g in (ngroups - 2, ngroups - 1):
        wait_out(g, g % 2)


def _gather_rows(tbl, cls_flat, api_flat, av, n_rows, d):
    rows_per_w = n_rows // NUM_WORKERS
    mesh = plsc.VectorSubcoreMesh(
        core_axis_name="c", subcore_axis_name="s",
        num_cores=NUM_CORES, num_subcores=NUM_SUBCORES)
    return pl.kernel(
        functools.partial(_gather_body, rows_per_w, av),
        out_type=jax.ShapeDtypeStruct((n_rows, d), jnp.float32),
        mesh=mesh,
        scratch_types=[
            pltpu.VMEM((2, NT), jnp.int32),
            pltpu.VMEM((2, NT), jnp.int32),
            pltpu.VMEM((2, G), jnp.int32),
            pltpu.VMEM((2, G, d), jnp.float32),
            pltpu.SemaphoreType.DMA,
            pltpu.SemaphoreType.DMA,
            pltpu.SemaphoreType.DMA,
            pltpu.SemaphoreType.DMA,
            pltpu.SemaphoreType.DMA,
            pltpu.SemaphoreType.DMA,
        ],
        compiler_params=pltpu.CompilerParams(use_tc_tiling_on_sc=False, needs_layout_passes=False),
    )(tbl, cls_flat, api_flat)


def kernel(class_seq, api_seq, class_table, api_table):
    b, s = class_seq.shape
    av, da = api_table.shape
    d_model = class_table.shape[1] + da
    scale = math.sqrt(float(d_model))
    t = b * s

    tbl = _build_table(api_table, class_table, scale)
    cls_flat = class_seq.reshape(t).astype(jnp.int32)
    api_flat = api_seq.reshape(t).astype(jnp.int32)
    out2 = _gather_rows(tbl, cls_flat, api_flat, av, 2 * t, da)
    return out2.reshape(b, s, d_model)


# speedup vs baseline: 13.7718x; 1.0007x over previous
import functools
import math

import jax
import jax.numpy as jnp
from jax import lax
from jax.experimental import pallas as pl
from jax.experimental.pallas import tpu as pltpu
from jax.experimental.pallas import tpu_sc as plsc

NUM_CORES = 2
NUM_SUBCORES = 16
NUM_WORKERS = NUM_CORES * NUM_SUBCORES
LANES = 16

CHUNK = 320
G = 640
NCH = G // CHUNK
NT = G // 2

SCALE_ROWS = 625


def _scale_body(scale, av, cv, api_hbm, cls_hbm, out_hbm, buf, zbuf, sd):
    wid = lax.axis_index("s") * NUM_CORES + lax.axis_index("c")

    def scale_buf(nrows):
        def body(i, carry):
            for j in range(64 // LANES):
                v = buf[i, pl.ds(j * LANES, LANES)]
                buf[i, pl.ds(j * LANES, LANES)] = v * scale
            return carry
        lax.fori_loop(0, nrows, body, 0)

    def do_span(src, src_base, dst_base, nrows):
        pltpu.async_copy(
            src.at[pl.ds(src_base, nrows)],
            buf.at[pl.ds(0, nrows), :], sd).wait()
        scale_buf(nrows)
        pltpu.async_copy(
            buf.at[pl.ds(0, nrows), :],
            out_hbm.at[pl.ds(dst_base, nrows)], sd).wait()

    def zero_row(dst_row):
        zero = jnp.zeros((LANES,), jnp.float32)
        for i in range(64 // LANES):
            zbuf[0, pl.ds(i * LANES, LANES)] = zero
        pltpu.async_copy(zbuf, out_hbm.at[pl.ds(dst_row, 1)], sd).wait()

    rows_w = av // NUM_WORKERS
    nch = rows_w // SCALE_ROWS
    base = wid * rows_w

    def api_chunk(c, carry):
        b = base + c * SCALE_ROWS
        do_span(api_hbm, b, b, SCALE_ROWS)
        return carry

    lax.fori_loop(0, nch, api_chunk, 0)
    rem = rows_w - nch * SCALE_ROWS
    if rem:
        do_span(api_hbm, base + nch * SCALE_ROWS, base + nch * SCALE_ROWS, rem)

    @pl.when(wid == 0)
    def _():
        zero_row(0)

    @pl.when(wid == NUM_WORKERS - 1)
    def _():
        nfull = cv // SCALE_ROWS
        for c in range(nfull):
            do_span(cls_hbm, c * SCALE_ROWS, av + c * SCALE_ROWS, SCALE_ROWS)
        crem = cv - nfull * SCALE_ROWS
        if crem:
            do_span(cls_hbm, nfull * SCALE_ROWS, av + nfull * SCALE_ROWS, crem)
        zero_row(av)


def _build_table(api_table, class_table, scale):
    av, d = api_table.shape
    cv = class_table.shape[0]
    mesh = plsc.VectorSubcoreMesh(
        core_axis_name="c", subcore_axis_name="s",
        num_cores=NUM_CORES, num_subcores=NUM_SUBCORES)
    return pl.kernel(
        functools.partial(_scale_body, scale, av, cv),
        out_type=jax.ShapeDtypeStruct((av + cv, d), jnp.float32),
        mesh=mesh,
        scratch_types=[
            pltpu.VMEM((SCALE_ROWS, d), jnp.float32),
            pltpu.VMEM((1, d), jnp.float32),
            pltpu.SemaphoreType.DMA,
        ],
        compiler_params=pltpu.CompilerParams(use_tc_tiling_on_sc=False, needs_layout_passes=False),
    )(api_table, class_table)


def _gather_body(rows_per_w, av, tbl_hbm, cls_hbm, api_hbm, out_hbm,
                 cls_v, api_v, idx_v, rows_v, si0, si1, sg0, sg1, so0, so1):
    wid = lax.axis_index("s") * NUM_CORES + lax.axis_index("c")
    w0 = wid * rows_per_w
    t0 = wid * (rows_per_w // 2)
    ngroups = rows_per_w // G
    si = (si0, si1)
    sg = (sg0, sg1)
    so = (so0, so1)
    pos0 = lax.iota(jnp.int32, LANES) * 2

    def idx_copy(g, s):
        tb = t0 + g * NT
        pltpu.async_copy(cls_hbm.at[pl.ds(tb, NT)], cls_v.at[s], si[s])
        pltpu.async_copy(api_hbm.at[pl.ds(tb, NT)], api_v.at[s], si[s])

    def interleave(s):
        def body(i, carry):
            c = cls_v[s, pl.ds(i * LANES, LANES)] + av
            a = api_v[s, pl.ds(i * LANES, LANES)]
            pos = pos0 + i * (2 * LANES)
            plsc.store_scatter(idx_v.at[s], [pos], c)
            plsc.store_scatter(idx_v.at[s], [pos + 1], a)
            return carry
        lax.fori_loop(0, NT // LANES, body, 0)

    def fire_group(g, s):
        tb = t0 + g * NT
        pltpu.make_async_copy(
            cls_hbm.at[pl.ds(tb, NT)], cls_v.at[s], si[s]).wait()
        pltpu.make_async_copy(
            api_hbm.at[pl.ds(tb, NT)], api_v.at[s], si[s]).wait()
        interleave(s)
        for j in range(NCH):
            pltpu.async_copy(
                tbl_hbm.at[idx_v.at[s, pl.ds(j * CHUNK, CHUNK)]],
                rows_v.at[s, pl.ds(j * CHUNK, CHUNK), :], sg[s])

    def drain_gathers(s):
        for j in range(NCH):
            pltpu.make_async_copy(
                tbl_hbm.at[idx_v.at[s, pl.ds(j * CHUNK, CHUNK)]],
                rows_v.at[s, pl.ds(j * CHUNK, CHUNK), :], sg[s]).wait()

    def fire_out(g, s):
        pltpu.async_copy(rows_v.at[s], out_hbm.at[pl.ds(w0 + g * G, G)], so[s])

    def wait_out(g, s):
        pltpu.make_async_copy(
            rows_v.at[s], out_hbm.at[pl.ds(w0 + g * G, G)], so[s]).wait()

    for s in range(2):
        idx_copy(s, s)
    fire_group(0, 0)
    idx_copy(2, 0)
    fire_group(1, 1)
    drain_gathers(0)
    fire_out(0, 0)
    idx_copy(3, 1)

    def outer(g2, carry):
        for s in range(2):
            g = 2 * g2 + s
            wait_out(g - 2, s)
            fire_group(g, s)
            drain_gathers(1 - s)
            fire_out(g - 1, 1 - s)
            idx_copy(g + 2, s)
        return carry

    lax.fori_loop(1, ngroups // 2 - 1, outer, 0)

    for g in (ngroups - 2, ngroups - 1):
        s = g % 2
        wait_out(g - 2, s)
        fire_group(g, s)
        drain_gathers(1 - s)
        fire_out(g - 1, 1 - s)
    drain_gathers((ngroups - 1) % 2)
    fire_out(ngroups - 1, (ngroups - 1) % 2)
    for g in (ngroups - 2, ngroups - 1):
        wait_out(g, g % 2)


def _gather_rows(tbl, cls_flat, api_flat, av, n_rows, d):
    rows_per_w = n_rows // NUM_WORKERS
    mesh = plsc.VectorSubcoreMesh(
        core_axis_name="c", subcore_axis_name="s",
        num_cores=NUM_CORES, num_subcores=NUM_SUBCORES)
    return pl.kernel(
        functools.partial(_gather_body, rows_per_w, av),
        out_type=jax.ShapeDtypeStruct((n_rows, d), jnp.float32),
        mesh=mesh,
        scratch_types=[
            pltpu.VMEM((2, NT), jnp.int32),
            pltpu.VMEM((2, NT), jnp.int32),
            pltpu.VMEM((2, G), jnp.int32),
            pltpu.VMEM((2, G, d), jnp.float32),
            pltpu.SemaphoreType.DMA,
            pltpu.SemaphoreType.DMA,
            pltpu.SemaphoreType.DMA,
            pltpu.SemaphoreType.DMA,
            pltpu.SemaphoreType.DMA,
            pltpu.SemaphoreType.DMA,
        ],
        compiler_params=pltpu.CompilerParams(use_tc_tiling_on_sc=False, needs_layout_passes=False),
    )(tbl, cls_flat, api_flat)


def kernel(class_seq, api_seq, class_table, api_table):
    b, s = class_seq.shape
    av, da = api_table.shape
    d_model = class_table.shape[1] + da
    scale = math.sqrt(float(d_model))
    t = b * s

    tbl = _build_table(api_table, class_table, scale)
    cls_flat = class_seq.reshape(t).astype(jnp.int32)
    api_flat = api_seq.reshape(t).astype(jnp.int32)
    out2 = _gather_rows(tbl, cls_flat, api_flat, av, 2 * t, da)
    return out2.reshape(b, s, d_model)


# speedup vs baseline: 13.8634x; 1.0067x over previous
import functools
import math

import jax
import jax.numpy as jnp
from jax import lax
from jax.experimental import pallas as pl
from jax.experimental.pallas import tpu as pltpu
from jax.experimental.pallas import tpu_sc as plsc

NUM_CORES = 2
NUM_SUBCORES = 16
NUM_WORKERS = NUM_CORES * NUM_SUBCORES
LANES = 16

NT = 400
SCALE_ROWS = 625


def _scale_body(scale, av, cv, api_hbm, cls_hbm, aout_hbm, cout_hbm,
                buf, zbuf, sd):
    wid = lax.axis_index("s") * NUM_CORES + lax.axis_index("c")

    def scale_buf(nrows):
        def body(i, carry):
            for j in range(64 // LANES):
                v = buf[i, pl.ds(j * LANES, LANES)]
                buf[i, pl.ds(j * LANES, LANES)] = v * scale
            return carry
        lax.fori_loop(0, nrows, body, 0)

    def do_span(src, dst, src_base, dst_base, nrows):
        pltpu.async_copy(
            src.at[pl.ds(src_base, nrows)],
            buf.at[pl.ds(0, nrows), :], sd).wait()
        scale_buf(nrows)
        pltpu.async_copy(
            buf.at[pl.ds(0, nrows), :],
            dst.at[pl.ds(dst_base, nrows)], sd).wait()

    def zero_row(dst, dst_row):
        zero = jnp.zeros((LANES,), jnp.float32)
        for i in range(64 // LANES):
            zbuf[0, pl.ds(i * LANES, LANES)] = zero
        pltpu.async_copy(zbuf, dst.at[pl.ds(dst_row, 1)], sd).wait()

    rows_w = av // NUM_WORKERS
    nch = rows_w // SCALE_ROWS
    base = wid * rows_w

    def api_chunk(c, carry):
        b = base + c * SCALE_ROWS
        do_span(api_hbm, aout_hbm, b, b, SCALE_ROWS)
        return carry

    lax.fori_loop(0, nch, api_chunk, 0)
    rem = rows_w - nch * SCALE_ROWS
    if rem:
        do_span(api_hbm, aout_hbm, base + nch * SCALE_ROWS,
                base + nch * SCALE_ROWS, rem)

    @pl.when(wid == 0)
    def _():
        zero_row(aout_hbm, 0)

    @pl.when(wid == NUM_WORKERS - 1)
    def _():
        nfull = cv // SCALE_ROWS
        for c in range(nfull):
            do_span(cls_hbm, cout_hbm, c * SCALE_ROWS, c * SCALE_ROWS,
                    SCALE_ROWS)
        crem = cv - nfull * SCALE_ROWS
        if crem:
            do_span(cls_hbm, cout_hbm, nfull * SCALE_ROWS,
                    nfull * SCALE_ROWS, crem)
        zero_row(cout_hbm, 0)


def _build_tables(api_table, class_table, scale):
    av, d = api_table.shape
    cv = class_table.shape[0]
    mesh = plsc.VectorSubcoreMesh(
        core_axis_name="c", subcore_axis_name="s",
        num_cores=NUM_CORES, num_subcores=NUM_SUBCORES)
    return pl.kernel(
        functools.partial(_scale_body, scale, av, cv),
        out_type=(jax.ShapeDtypeStruct((av, d), jnp.float32),
                  jax.ShapeDtypeStruct((cv, d), jnp.float32)),
        mesh=mesh,
        scratch_types=[
            pltpu.VMEM((SCALE_ROWS, d), jnp.float32),
            pltpu.VMEM((1, d), jnp.float32),
            pltpu.SemaphoreType.DMA,
        ],
        compiler_params=pltpu.CompilerParams(
            use_tc_tiling_on_sc=False, needs_layout_passes=False),
    )(api_table, class_table)


def _gather_body(tpw, atbl, ctbl, cls_hbm, api_hbm, out_hbm,
                 cidx_v, aidx_v, crows, arows, si0, si1, sg0, sg1, so0, so1):
    wid = lax.axis_index("s") * NUM_CORES + lax.axis_index("c")
    t0 = wid * tpw
    ngroups = tpw // NT
    si = (si0, si1)
    sg = (sg0, sg1)
    so = (so0, so1)

    def idx_copy(g, s):
        tb = t0 + g * NT
        pltpu.async_copy(cls_hbm.at[pl.ds(tb, NT)], cidx_v.at[s], si[s])
        pltpu.async_copy(api_hbm.at[pl.ds(tb, NT)], aidx_v.at[s], si[s])

    def fire_group(g, s):
        tb = t0 + g * NT
        pltpu.make_async_copy(
            cls_hbm.at[pl.ds(tb, NT)], cidx_v.at[s], si[s]).wait()
        pltpu.make_async_copy(
            api_hbm.at[pl.ds(tb, NT)], aidx_v.at[s], si[s]).wait()
        pltpu.async_copy(ctbl.at[cidx_v.at[s]], crows.at[s], sg[s])
        pltpu.async_copy(atbl.at[aidx_v.at[s]], arows.at[s], sg[s])

    def drain_gathers(s):
        pltpu.make_async_copy(ctbl.at[cidx_v.at[s]], crows.at[s], sg[s]).wait()
        pltpu.make_async_copy(atbl.at[aidx_v.at[s]], arows.at[s], sg[s]).wait()

    def fire_out(g, s):
        tb = t0 + g * NT
        pltpu.async_copy(crows.at[s], out_hbm.at[pl.ds(tb, NT), 0, :], so[s])
        pltpu.async_copy(arows.at[s], out_hbm.at[pl.ds(tb, NT), 1, :], so[s])

    def wait_out(g, s):
        tb = t0 + g * NT
        pltpu.make_async_copy(
            crows.at[s], out_hbm.at[pl.ds(tb, NT), 0, :], so[s]).wait()
        pltpu.make_async_copy(
            arows.at[s], out_hbm.at[pl.ds(tb, NT), 1, :], so[s]).wait()

    for s in range(2):
        idx_copy(s, s)
    fire_group(0, 0)
    idx_copy(2, 0)
    fire_group(1, 1)
    drain_gathers(0)
    fire_out(0, 0)
    idx_copy(3, 1)

    def outer(g2, carry):
        for s in range(2):
            g = 2 * g2 + s
            wait_out(g - 2, s)
            fire_group(g, s)
            drain_gathers(1 - s)
            fire_out(g - 1, 1 - s)
            idx_copy(g + 2, s)
        return carry

    lax.fori_loop(1, ngroups // 2 - 1, outer, 0)

    for g in (ngroups - 2, ngroups - 1):
        s = g % 2
        wait_out(g - 2, s)
        fire_group(g, s)
        drain_gathers(1 - s)
        fire_out(g - 1, 1 - s)
    drain_gathers((ngroups - 1) % 2)
    fire_out(ngroups - 1, (ngroups - 1) % 2)
    for g in (ngroups - 2, ngroups - 1):
        wait_out(g, g % 2)


def _gather_rows(atbl, ctbl, cls_flat, api_flat, n_tok, d):
    tpw = n_tok // NUM_WORKERS
    mesh = plsc.VectorSubcoreMesh(
        core_axis_name="c", subcore_axis_name="s",
        num_cores=NUM_CORES, num_subcores=NUM_SUBCORES)
    return pl.kernel(
        functools.partial(_gather_body, tpw),
        out_type=jax.ShapeDtypeStruct((n_tok, 2, d), jnp.float32),
        mesh=mesh,
        scratch_types=[
            pltpu.VMEM((2, NT), jnp.int32),
            pltpu.VMEM((2, NT), jnp.int32),
            pltpu.VMEM((2, NT, d), jnp.float32),
            pltpu.VMEM((2, NT, d), jnp.float32),
            pltpu.SemaphoreType.DMA,
            pltpu.SemaphoreType.DMA,
            pltpu.SemaphoreType.DMA,
            pltpu.SemaphoreType.DMA,
            pltpu.SemaphoreType.DMA,
            pltpu.SemaphoreType.DMA,
        ],
        compiler_params=pltpu.CompilerParams(
            use_tc_tiling_on_sc=False, needs_layout_passes=False),
    )(atbl, ctbl, cls_flat, api_flat)


def kernel(class_seq, api_seq, class_table, api_table):
    b, s = class_seq.shape
    av, da = api_table.shape
    d_model = class_table.shape[1] + da
    scale = math.sqrt(float(d_model))
    t = b * s

    atbl, ctbl = _build_tables(api_table, class_table, scale)
    cls_flat = class_seq.reshape(t).astype(jnp.int32)
    api_flat = api_seq.reshape(t).astype(jnp.int32)
    out3 = _gather_rows(atbl, ctbl, cls_flat, api_flat, t, da)
    return out3.reshape(b, s, d_model)
